# Initial kernel scaffold; baseline (speedup 1.0000x reference)
import jax
import jax.numpy as jnp
from jax.experimental import pallas as pl


def kernel(inputs, weight):
    raise NotImplementedError("write your pallas kernel here")


# speedup vs baseline: 1.6005x; 1.6005x over previous
import functools

import jax
import jax.numpy as jnp
from jax import lax
from jax.experimental import pallas as pl
from jax.experimental.pallas import tpu as pltpu
from jax.experimental.pallas import tpu_sc as plsc

B = 16
C = 256
HW = 1024
NE = 8192
NT = 8
TN = NE // NT
NPOS = B * HW
COMMITMENT_COST = 0.25

NC = 2
NS = 16
L = 16
NW = NC * NS
BPW = NPOS // NW
CH = 128
NCH = BPW // CH


def _argmin_body(x_ref, w_ref, idx_ref, acc_ref):
    b = pl.program_id(0)
    x = x_ref[0]
    xsq = jnp.sum(x * x, axis=0, keepdims=True)
    best_val = jnp.full((1, HW), jnp.inf, jnp.float32)
    best_idx = jnp.zeros((1, HW), jnp.int32)
    for j in range(NT):
        w = w_ref[pl.ds(j * TN, TN), :]
        wsq = jnp.sum(w * w, axis=1, keepdims=True)
        dot = lax.dot_general(w.astype(jnp.bfloat16), x.astype(jnp.bfloat16),
                              (((1,), (0,)), ((), ())),
                              preferred_element_type=jnp.float32)
        scores = (xsq + wsq) - 2.0 * dot
        m = jnp.min(scores, axis=0, keepdims=True)
        iota = lax.broadcasted_iota(jnp.int32, (TN, HW), 0)
        cand = jnp.where(scores == m, iota, NE)
        mi = jnp.min(cand, axis=0, keepdims=True) + j * TN
        upd = m < best_val
        best_idx = jnp.where(upd, mi, best_idx)
        best_val = jnp.where(upd, m, best_val).astype(jnp.bfloat16).astype(jnp.float32)
    idx_ref[0] = best_idx

    @pl.when(b == 0)
    def _():
        acc_ref[...] = jnp.zeros((1, 1), jnp.float32)

    acc_ref[...] += jnp.sum(best_val, keepdims=True)


def _gather_hist_body(w_hbm, idx_hbm, out_hbm, cnt_hbm,
                      idx_v, rows_v, ones_v, zer_v, cnt_sh, sem):
    c = lax.axis_index("c")
    s = lax.axis_index("s")
    wid = s * NC + c

    pltpu.sync_copy(idx_hbm.at[pl.ds(wid * NCH, NCH)], idx_v)

    for t in range(CH // L):
        ones_v[pl.ds(t * L, L)] = jnp.full((L,), 1.0, jnp.float32)

    @pl.when(s == 0)
    def _():
        def zbody(i, carry):
            zer_v[pl.ds(i * L, L)] = jnp.zeros((L,), jnp.float32)
            return carry
        lax.fori_loop(0, NE // L, zbody, 0)
        pltpu.sync_copy(zer_v, cnt_sh)

    plsc.subcore_barrier()

    for j in range(NCH):
        pltpu.async_copy(w_hbm.at[idx_v.at[j]], rows_v, sem).wait()
        pltpu.sync_copy(rows_v, out_hbm.at[pl.ds(wid * BPW + j * CH, CH)])
        pltpu.sync_copy(ones_v, cnt_sh.at[idx_v.at[j]], add=True)

    plsc.subcore_barrier()

    @pl.when(s == 0)
    def _():
        pltpu.sync_copy(cnt_sh, cnt_hbm.at[c])


def _finalize_body(acc_ref, cnt_ref, loss_ref, plex_ref):
    counts = cnt_ref[0:1, :] + cnt_ref[1:2, :]
    p = counts * (1.0 / NPOS)
    ent = jnp.sum(p * jnp.log(p + 1e-10), keepdims=True)
    plex_ref[...] = jnp.exp(-ent)
    loss_ref[...] = COMMITMENT_COST * (acc_ref[...] / (NPOS * C))


def _run_argmin(xr, weight):
    return pl.pallas_call(
        _argmin_body,
        grid=(B,),
        in_specs=[
            pl.BlockSpec((1, C, HW), lambda b: (b, 0, 0)),
            pl.BlockSpec((NE, C), lambda b: (0, 0)),
        ],
        out_specs=[
            pl.BlockSpec((1, 1, HW), lambda b: (b, 0, 0)),
            pl.BlockSpec((1, 1), lambda b: (0, 0)),
        ],
        out_shape=[
            jax.ShapeDtypeStruct((B, 1, HW), jnp.int32),
            jax.ShapeDtypeStruct((1, 1), jnp.float32),
        ],
        compiler_params=pltpu.CompilerParams(
            dimension_semantics=("arbitrary",)),
    )(xr, weight)


@functools.cache
def _make_gather_hist():
    return pl.kernel(
        _gather_hist_body,
        out_type=[
            jax.ShapeDtypeStruct((NPOS, C), jnp.float32),
            jax.ShapeDtypeStruct((NC, NE), jnp.float32),
        ],
        mesh=plsc.VectorSubcoreMesh(core_axis_name="c", subcore_axis_name="s",
                                    num_cores=NC, num_subcores=NS),
        scratch_types=[
            pltpu.VMEM((NCH, CH), jnp.int32),
            pltpu.VMEM((CH, C), jnp.float32),
            pltpu.VMEM((CH,), jnp.float32),
            pltpu.VMEM((NE,), jnp.float32),
            pltpu.VMEM_SHARED((NE,), jnp.float32),
            pltpu.SemaphoreType.DMA,
        ],
    )


def _run_finalize(acc, counts):
    return pl.pallas_call(
        _finalize_body,
        out_shape=[
            jax.ShapeDtypeStruct((1, 1), jnp.float32),
            jax.ShapeDtypeStruct((1, 1), jnp.float32),
        ],
    )(acc, counts)


def kernel(inputs, weight):
    xr = inputs.reshape(B, C, HW)
    idx3, acc = _run_argmin(xr, weight)
    idx2 = idx3.reshape(NW * NCH, CH)
    quant, counts = _make_gather_hist()(weight, idx2)
    loss, plex = _run_finalize(acc, counts)
    q = quant.reshape(B, 32, 32, C).transpose(0, 3, 1, 2)
    idx_out = idx3.reshape(B, 32, 32)
    return loss.reshape(()), q, plex.reshape(()), idx_out
